# baseline (speedup 1.0000x reference)
import functools

import jax
import jax.numpy as jnp
from jax.experimental import pallas as pl
from jax.experimental.pallas import tpu as pltpu

_TM0 = 128
_TC = 2048
_TX1 = 1024
_TF = 128


def _ccxn_kernel(x0_ref, w0_ref, a_ref, x1_ref, w12_ref, inc_ref,
                 oT_ref, o2_ref, aT_ref, m0T_ref, m1_ref,
                 *, nt0, nc, nx1, ntf):
    c = pl.program_id(0)
    i = pl.program_id(1)

    @pl.when(jnp.logical_and(c == 0, i == 0))
    def _():
        m0T = jax.lax.dot_general(
            w0_ref[0], x0_ref[...], (((0,), (1,)), ((), ())),
            preferred_element_type=jnp.float32)
        m0T_ref[...] = m0T.astype(jnp.bfloat16)

    @pl.when(jnp.logical_and(c == 0, i < nt0))
    def _():
        col = pl.multiple_of(i * _TM0, _TM0)
        a_bf = a_ref[...].astype(jnp.bfloat16)
        aT = jnp.swapaxes(a_bf, 0, 1)
        aT_ref[:, pl.ds(col, _TM0)] = aT
        h = jax.lax.dot_general(m0T_ref[...], aT, (((1,), (0,)), ((), ())),
                                preferred_element_type=jnp.float32)
        oT_ref[0, :, pl.ds(col, _TM0)] = jnp.maximum(h, 0.0)

    in_l12 = jnp.logical_and(c == 0,
                             jnp.logical_and(i >= nt0, i < nt0 + 2 * nc))

    @pl.when(jnp.logical_and(in_l12, (i - nt0) % nc == 0))
    def _():
        m0T = jax.lax.dot_general(
            w0_ref[0], oT_ref[0], (((0,), (0,)), ((), ())),
            preferred_element_type=jnp.float32)
        m0T_ref[...] = m0T.astype(jnp.bfloat16)

    @pl.when(in_l12)
    def _():
        col = pl.multiple_of(((i - nt0) % nc) * _TC, _TC)
        h = jax.lax.dot_general(m0T_ref[...], aT_ref[:, pl.ds(col, _TC)],
                                (((1,), (0,)), ((), ())),
                                preferred_element_type=jnp.float32)
        oT_ref[0, :, pl.ds(col, _TC)] = jnp.maximum(h, 0.0)

    @pl.when(jnp.logical_and(c == 1, i < nx1))
    def _():
        row = pl.multiple_of(i * _TX1, _TX1)
        m1 = jnp.dot(x1_ref[...].astype(jnp.bfloat16),
                     w12_ref[...].astype(jnp.bfloat16),
                     preferred_element_type=jnp.float32)
        m1_ref[pl.ds(row, _TX1), :] = m1.astype(jnp.bfloat16)

    @pl.when(jnp.logical_and(c == 1, i >= nx1))
    def _():
        h = jnp.dot(inc_ref[...].astype(jnp.bfloat16), m1_ref[...],
                    preferred_element_type=jnp.float32)
        o2_ref[0] = jnp.maximum(h, 0.0)


def kernel(x_0, x_1, adjacency_0, incidence_2_t, w0_stack, w12_stack):
    n_nodes, c0 = x_0.shape
    n_edges, c1 = x_1.shape
    n_faces = incidence_2_t.shape[0]
    n_layers = w0_stack.shape[0]
    c2 = w12_stack.shape[2]

    nt0 = n_nodes // _TM0
    nc = n_nodes // _TC
    nx1 = n_edges // _TX1
    ntf = n_faces // _TF
    n_node_steps = nt0 + (n_layers - 1) * nc
    n_steps = max(n_node_steps, nx1 + ntf)

    x0T_out, x2_out = pl.pallas_call(
        functools.partial(_ccxn_kernel, nt0=nt0, nc=nc, nx1=nx1, ntf=ntf),
        grid=(2, n_steps),
        out_shape=(
            jax.ShapeDtypeStruct((2, c0, n_nodes), x_0.dtype),
            jax.ShapeDtypeStruct((2, n_faces, c2), x_1.dtype),
        ),
        in_specs=[
            pl.BlockSpec((n_nodes, c0), lambda c, i: (0, 0)),
            pl.BlockSpec(
                (1, c0, c0),
                lambda c, i: (jnp.where(
                    jnp.logical_or(c == 1, i < nt0), 0,
                    jnp.minimum((i - nt0) // nc + 1, 2)), 0, 0)),
            pl.BlockSpec(
                (_TM0, n_nodes),
                lambda c, i: (jnp.where(c == 0, jnp.minimum(i, nt0 - 1), 0), 0)),
            pl.BlockSpec(
                (_TX1, c1),
                lambda c, i: (jnp.where(c == 1, jnp.minimum(i, nx1 - 1), 0), 0)),
            pl.BlockSpec((c1, c2), lambda c, i: (0, 0)),
            pl.BlockSpec(
                (_TF, n_edges),
                lambda c, i: (jnp.where(
                    c == 1, jnp.clip(i - nx1, 0, ntf - 1), 0), 0)),
        ],
        out_specs=(
            pl.BlockSpec((1, c0, n_nodes), lambda c, i: (c, 0, 0)),
            pl.BlockSpec(
                (1, _TF, c2),
                lambda c, i: (c, jnp.clip(i - nx1, 0, ntf - 1), 0)),
        ),
        scratch_shapes=[
            pltpu.VMEM((n_nodes, n_nodes), jnp.bfloat16),
            pltpu.VMEM((c0, n_nodes), jnp.bfloat16),
            pltpu.VMEM((n_edges, c2), jnp.bfloat16),
        ],
        compiler_params=pltpu.CompilerParams(
            dimension_semantics=("parallel", "arbitrary")),
    )(x_0, w0_stack, adjacency_0, x_1, w12_stack[n_layers - 1], incidence_2_t)

    return jnp.transpose(x0T_out[0]), x_1, x2_out[1]


# speedup vs baseline: 1.4293x; 1.4293x over previous
import functools

import jax
import jax.numpy as jnp
from jax.experimental import pallas as pl
from jax.experimental.pallas import tpu as pltpu

_TM0 = 256
_TC = 2048
_TX1 = 1024
_TF = 128


def _face_tile(rel):
    return rel - 1 - rel // 5


def _ccxn_kernel(x0_ref, w0_ref, a_ref, x1_ref, w12_ref, inc_ref,
                 oT_ref, o2_ref, aT_ref, m0T_ref, m1_ref,
                 *, nt0, nc, nx1):
    s = pl.program_id(0)

    @pl.when(s == 0)
    def _():
        m0T = jax.lax.dot_general(
            w0_ref[0], x0_ref[...], (((0,), (1,)), ((), ())),
            preferred_element_type=jnp.float32)
        m0T_ref[...] = m0T.astype(jnp.bfloat16)

    @pl.when(s < nt0)
    def _():
        col = pl.multiple_of(s * _TM0, _TM0)
        a_bf = a_ref[...].astype(jnp.bfloat16)
        aT = jnp.swapaxes(a_bf, 0, 1)
        aT_ref[:, pl.ds(col, _TM0)] = aT
        h = jax.lax.dot_general(m0T_ref[...], aT, (((1,), (0,)), ((), ())),
                                preferred_element_type=jnp.float32)
        oT_ref[:, pl.ds(col, _TM0)] = jnp.maximum(h, 0.0)

    @pl.when(s < nx1)
    def _():
        row = pl.multiple_of(s * _TX1, _TX1)
        m1 = jnp.dot(x1_ref[...].astype(jnp.bfloat16),
                     w12_ref[...].astype(jnp.bfloat16),
                     preferred_element_type=jnp.float32)
        m1_ref[pl.ds(row, _TX1), :] = m1.astype(jnp.bfloat16)

    rel = s - nt0
    is_chunk = jnp.logical_and(s >= nt0, rel % 5 == 0)
    is_face = jnp.logical_and(s >= nt0, rel % 5 != 0)

    @pl.when(jnp.logical_and(is_chunk, rel % (5 * nc) == 0))
    def _():
        m0T = jax.lax.dot_general(
            w0_ref[0], oT_ref[...], (((0,), (0,)), ((), ())),
            preferred_element_type=jnp.float32)
        m0T_ref[...] = m0T.astype(jnp.bfloat16)

    @pl.when(is_chunk)
    def _():
        col = pl.multiple_of(((rel // 5) % nc) * _TC, _TC)
        h = jax.lax.dot_general(m0T_ref[...], aT_ref[:, pl.ds(col, _TC)],
                                (((1,), (0,)), ((), ())),
                                preferred_element_type=jnp.float32)
        oT_ref[:, pl.ds(col, _TC)] = jnp.maximum(h, 0.0)

    @pl.when(is_face)
    def _():
        h = jnp.dot(inc_ref[...].astype(jnp.bfloat16), m1_ref[...],
                    preferred_element_type=jnp.float32)
        o2_ref[...] = jnp.maximum(h, 0.0)


def kernel(x_0, x_1, adjacency_0, incidence_2_t, w0_stack, w12_stack):
    n_nodes, c0 = x_0.shape
    n_edges, c1 = x_1.shape
    n_faces = incidence_2_t.shape[0]
    n_layers = w0_stack.shape[0]
    c2 = w12_stack.shape[2]

    nt0 = n_nodes // _TM0
    nc = n_nodes // _TC
    nx1 = n_edges // _TX1
    ntf = n_faces // _TF
    np2 = (n_layers - 1) * nc + ntf
    n_steps = nt0 + np2

    def _w0_idx(s):
        rel = jnp.maximum(s - nt0, 0)
        return jnp.where(s < nt0, 0, jnp.minimum(rel // (5 * nc) + 1, n_layers - 1))

    def _inc_idx(s):
        rel = jnp.maximum(s - nt0, 0)
        return jnp.clip(_face_tile(rel), 0, ntf - 1)

    x0T_out, x2_out = pl.pallas_call(
        functools.partial(_ccxn_kernel, nt0=nt0, nc=nc, nx1=nx1),
        grid=(n_steps,),
        out_shape=(
            jax.ShapeDtypeStruct((c0, n_nodes), x_0.dtype),
            jax.ShapeDtypeStruct((n_faces, c2), x_1.dtype),
        ),
        in_specs=[
            pl.BlockSpec((n_nodes, c0), lambda s: (0, 0)),
            pl.BlockSpec((1, c0, c0), lambda s: (_w0_idx(s), 0, 0)),
            pl.BlockSpec((_TM0, n_nodes),
                         lambda s: (jnp.minimum(s, nt0 - 1), 0)),
            pl.BlockSpec((_TX1, c1),
                         lambda s: (jnp.minimum(s, nx1 - 1), 0)),
            pl.BlockSpec((c1, c2), lambda s: (0, 0)),
            pl.BlockSpec((_TF, n_edges), lambda s: (_inc_idx(s), 0)),
        ],
        out_specs=(
            pl.BlockSpec((c0, n_nodes), lambda s: (0, 0)),
            pl.BlockSpec((_TF, c2), lambda s: (_inc_idx(s), 0)),
        ),
        scratch_shapes=[
            pltpu.VMEM((n_nodes, n_nodes), jnp.bfloat16),
            pltpu.VMEM((c0, n_nodes), jnp.bfloat16),
            pltpu.VMEM((n_edges, c2), jnp.bfloat16),
        ],
        compiler_params=pltpu.CompilerParams(
            dimension_semantics=("arbitrary",)),
    )(x_0, w0_stack, adjacency_0, x_1, w12_stack[n_layers - 1], incidence_2_t)

    return jnp.transpose(x0T_out), x_1, x2_out


# speedup vs baseline: 1.6160x; 1.1306x over previous
import functools

import jax
import jax.numpy as jnp
from jax.experimental import pallas as pl
from jax.experimental.pallas import tpu as pltpu

_TM0 = 512
_TC = 2048
_TX1 = 1024
_TF = 256


def _face_tile(rel):
    return rel - 1 - rel // 3


def _ccxn_kernel(x0_ref, w0_ref, a_ref, x1_ref, w12_ref, inc_ref,
                 oT_ref, o2_ref, aT_ref, m0T_ref, m1_ref,
                 *, nt0, nc, nx1):
    s = pl.program_id(0)

    @pl.when(s == 0)
    def _():
        m0T = jax.lax.dot_general(
            w0_ref[0], x0_ref[...], (((0,), (1,)), ((), ())),
            preferred_element_type=jnp.float32)
        m0T_ref[...] = m0T.astype(jnp.bfloat16)

    @pl.when(s < nt0)
    def _():
        col = pl.multiple_of(s * _TM0, _TM0)
        a_bf = a_ref[...].astype(jnp.bfloat16)
        aT = jnp.swapaxes(a_bf, 0, 1)
        aT_ref[:, pl.ds(col, _TM0)] = aT.astype(jnp.int8)
        h = jax.lax.dot_general(m0T_ref[...], aT, (((1,), (0,)), ((), ())),
                                preferred_element_type=jnp.float32)
        oT_ref[:, pl.ds(col, _TM0)] = jnp.maximum(h, 0.0)

    @pl.when(s < nx1)
    def _():
        row = pl.multiple_of(s * _TX1, _TX1)
        m1 = jnp.dot(x1_ref[...].astype(jnp.bfloat16),
                     w12_ref[...].astype(jnp.bfloat16),
                     preferred_element_type=jnp.float32)
        m1_ref[pl.ds(row, _TX1), :] = m1.astype(jnp.bfloat16)

    rel = s - nt0
    is_chunk = jnp.logical_and(s >= nt0, rel % 3 == 0)
    is_face = jnp.logical_and(s >= nt0, rel % 3 != 0)

    @pl.when(jnp.logical_and(is_chunk, rel % (3 * nc) == 0))
    def _():
        m0T = jax.lax.dot_general(
            w0_ref[0], oT_ref[...], (((0,), (0,)), ((), ())),
            preferred_element_type=jnp.float32)
        m0T_ref[...] = m0T.astype(jnp.bfloat16)

    @pl.when(is_chunk)
    def _():
        col = pl.multiple_of(((rel // 3) % nc) * _TC, _TC)
        a_chunk = aT_ref[:, pl.ds(col, _TC)].astype(jnp.bfloat16)
        h = jax.lax.dot_general(m0T_ref[...], a_chunk,
                                (((1,), (0,)), ((), ())),
                                preferred_element_type=jnp.float32)
        oT_ref[:, pl.ds(col, _TC)] = jnp.maximum(h, 0.0)

    @pl.when(is_face)
    def _():
        h = jnp.dot(inc_ref[...].astype(jnp.bfloat16), m1_ref[...],
                    preferred_element_type=jnp.float32)
        o2_ref[...] = jnp.maximum(h, 0.0)


def kernel(x_0, x_1, adjacency_0, incidence_2_t, w0_stack, w12_stack):
    n_nodes, c0 = x_0.shape
    n_edges, c1 = x_1.shape
    n_faces = incidence_2_t.shape[0]
    n_layers = w0_stack.shape[0]
    c2 = w12_stack.shape[2]

    nt0 = n_nodes // _TM0
    nc = n_nodes // _TC
    nx1 = n_edges // _TX1
    ntf = n_faces // _TF
    np2 = (n_layers - 1) * nc + ntf
    n_steps = nt0 + np2

    def _w0_idx(s):
        rel = jnp.maximum(s - nt0, 0)
        return jnp.where(s < nt0, 0,
                         jnp.minimum(rel // (3 * nc) + 1, n_layers - 1))

    def _inc_idx(s):
        rel = jnp.maximum(s - nt0, 0)
        return jnp.clip(_face_tile(rel), 0, ntf - 1)

    x0T_out, x2_out = pl.pallas_call(
        functools.partial(_ccxn_kernel, nt0=nt0, nc=nc, nx1=nx1),
        grid=(n_steps,),
        out_shape=(
            jax.ShapeDtypeStruct((c0, n_nodes), x_0.dtype),
            jax.ShapeDtypeStruct((n_faces, c2), x_1.dtype),
        ),
        in_specs=[
            pl.BlockSpec((n_nodes, c0), lambda s: (0, 0)),
            pl.BlockSpec((1, c0, c0), lambda s: (_w0_idx(s), 0, 0)),
            pl.BlockSpec((_TM0, n_nodes),
                         lambda s: (jnp.minimum(s, nt0 - 1), 0)),
            pl.BlockSpec((_TX1, c1),
                         lambda s: (jnp.minimum(s, nx1 - 1), 0)),
            pl.BlockSpec((c1, c2), lambda s: (0, 0)),
            pl.BlockSpec((_TF, n_edges), lambda s: (_inc_idx(s), 0)),
        ],
        out_specs=(
            pl.BlockSpec((c0, n_nodes), lambda s: (0, 0)),
            pl.BlockSpec((_TF, c2), lambda s: (_inc_idx(s), 0)),
        ),
        scratch_shapes=[
            pltpu.VMEM((n_nodes, n_nodes), jnp.int8),
            pltpu.VMEM((c0, n_nodes), jnp.bfloat16),
            pltpu.VMEM((n_edges, c2), jnp.bfloat16),
        ],
        compiler_params=pltpu.CompilerParams(
            dimension_semantics=("arbitrary",)),
    )(x_0, w0_stack, adjacency_0, x_1, w12_stack[n_layers - 1], incidence_2_t)

    return jnp.transpose(x0T_out), x_1, x2_out


# speedup vs baseline: 1.6810x; 1.0402x over previous
import functools

import jax
import jax.numpy as jnp
from jax.experimental import pallas as pl
from jax.experimental.pallas import tpu as pltpu

_TM0 = 512
_TC = 2048
_TX1 = 1024
_TF = 256


def _face_tile(rel):
    return rel - 1 - rel // 3


def _ccxn_kernel(x0_ref, w0_ref, a_ref, x1_ref, w12_ref, inc_ref,
                 o1_ref, o2_ref, aT_ref, sT_ref, m0T_ref, m1_ref,
                 *, nt0, nc, nx1):
    s = pl.program_id(0)

    @pl.when(s == 0)
    def _():
        m0T = jax.lax.dot_general(
            w0_ref[0], x0_ref[...], (((0,), (1,)), ((), ())),
            preferred_element_type=jnp.float32)
        m0T_ref[...] = m0T.astype(jnp.bfloat16)

    @pl.when(s < nt0)
    def _():
        col = pl.multiple_of(s * _TM0, _TM0)
        a_bf = a_ref[...].astype(jnp.bfloat16)
        aT = jnp.swapaxes(a_bf, 0, 1)
        aT_ref[:, pl.ds(col, _TM0)] = aT.astype(jnp.int8)
        h = jax.lax.dot_general(m0T_ref[...], aT, (((1,), (0,)), ((), ())),
                                preferred_element_type=jnp.float32)
        sT_ref[:, pl.ds(col, _TM0)] = jnp.maximum(h, 0.0).astype(jnp.bfloat16)

    @pl.when(s < nx1)
    def _():
        row = pl.multiple_of(s * _TX1, _TX1)
        m1 = jnp.dot(x1_ref[...].astype(jnp.bfloat16),
                     w12_ref[...].astype(jnp.bfloat16),
                     preferred_element_type=jnp.float32)
        m1_ref[pl.ds(row, _TX1), :] = m1.astype(jnp.bfloat16)

    rel = s - nt0
    is_chunk = jnp.logical_and(s >= nt0, rel % 3 == 0)
    is_face = jnp.logical_and(s >= nt0, rel % 3 != 0)

    @pl.when(jnp.logical_and(is_chunk, rel % (3 * nc) == 0))
    def _():
        m0T = jax.lax.dot_general(
            w0_ref[0], sT_ref[...], (((0,), (0,)), ((), ())),
            preferred_element_type=jnp.float32)
        m0T_ref[...] = m0T.astype(jnp.bfloat16)

    @pl.when(jnp.logical_and(is_chunk, rel < 3 * nc))
    def _():
        col = pl.multiple_of(((rel // 3) % nc) * _TC, _TC)
        a_chunk = aT_ref[:, pl.ds(col, _TC)].astype(jnp.bfloat16)
        h = jax.lax.dot_general(m0T_ref[...], a_chunk,
                                (((1,), (0,)), ((), ())),
                                preferred_element_type=jnp.float32)
        sT_ref[:, pl.ds(col, _TC)] = jnp.maximum(h, 0.0).astype(jnp.bfloat16)

    @pl.when(jnp.logical_and(is_chunk, rel >= 3 * nc))
    def _():
        col = pl.multiple_of(((rel // 3) % nc) * _TC, _TC)
        a_chunk = aT_ref[:, pl.ds(col, _TC)].astype(jnp.bfloat16)
        h = jax.lax.dot_general(m0T_ref[...], a_chunk,
                                (((1,), (0,)), ((), ())),
                                preferred_element_type=jnp.float32)
        o1_ref[...] = jnp.swapaxes(jnp.maximum(h, 0.0), 0, 1)

    @pl.when(is_face)
    def _():
        h = jnp.dot(inc_ref[...].astype(jnp.bfloat16), m1_ref[...],
                    preferred_element_type=jnp.float32)
        o2_ref[...] = jnp.maximum(h, 0.0)


def kernel(x_0, x_1, adjacency_0, incidence_2_t, w0_stack, w12_stack):
    n_nodes, c0 = x_0.shape
    n_edges, c1 = x_1.shape
    n_faces = incidence_2_t.shape[0]
    n_layers = w0_stack.shape[0]
    c2 = w12_stack.shape[2]

    nt0 = n_nodes // _TM0
    nc = n_nodes // _TC
    nx1 = n_edges // _TX1
    ntf = n_faces // _TF
    np2 = (n_layers - 1) * nc + ntf
    n_steps = nt0 + np2

    def _w0_idx(s):
        rel = jnp.maximum(s - nt0, 0)
        return jnp.where(s < nt0, 0,
                         jnp.minimum(rel // (3 * nc) + 1, n_layers - 1))

    def _inc_idx(s):
        rel = jnp.maximum(s - nt0, 0)
        return jnp.clip(_face_tile(rel), 0, ntf - 1)

    def _o1_idx(s):
        rel = jnp.maximum(s - nt0, 0)
        return jnp.clip((rel - 3 * nc) // 3, 0, nc - 1)

    x0_out, x2_out = pl.pallas_call(
        functools.partial(_ccxn_kernel, nt0=nt0, nc=nc, nx1=nx1),
        grid=(n_steps,),
        out_shape=(
            jax.ShapeDtypeStruct((n_nodes, c0), x_0.dtype),
            jax.ShapeDtypeStruct((n_faces, c2), x_1.dtype),
        ),
        in_specs=[
            pl.BlockSpec((n_nodes, c0), lambda s: (0, 0)),
            pl.BlockSpec((1, c0, c0), lambda s: (_w0_idx(s), 0, 0)),
            pl.BlockSpec((_TM0, n_nodes),
                         lambda s: (jnp.minimum(s, nt0 - 1), 0)),
            pl.BlockSpec((_TX1, c1),
                         lambda s: (jnp.minimum(s, nx1 - 1), 0)),
            pl.BlockSpec((c1, c2), lambda s: (0, 0)),
            pl.BlockSpec((_TF, n_edges), lambda s: (_inc_idx(s), 0)),
        ],
        out_specs=(
            pl.BlockSpec((_TC, c0), lambda s: (_o1_idx(s), 0)),
            pl.BlockSpec((_TF, c2), lambda s: (_inc_idx(s), 0)),
        ),
        scratch_shapes=[
            pltpu.VMEM((n_nodes, n_nodes), jnp.int8),
            pltpu.VMEM((c0, n_nodes), jnp.bfloat16),
            pltpu.VMEM((c0, n_nodes), jnp.bfloat16),
            pltpu.VMEM((n_edges, c2), jnp.bfloat16),
        ],
        compiler_params=pltpu.CompilerParams(
            dimension_semantics=("arbitrary",)),
    )(x_0, w0_stack, adjacency_0, x_1, w12_stack[n_layers - 1], incidence_2_t)

    return x0_out, x_1, x2_out


# speedup vs baseline: 1.7371x; 1.0334x over previous
import functools

import jax
import jax.numpy as jnp
from jax.experimental import pallas as pl
from jax.experimental.pallas import tpu as pltpu

_TM0 = 512
_TC = 2048
_TX1 = 1024
_TF = 256


def _ccxn_kernel(x0_ref, w0_ref, a_ref, x1_ref, w12_ref, inc_ref,
                 o1_ref, o2_ref, aT_ref, sT_ref, m0T_ref, m1_ref,
                 *, nt0, nc, nx1):
    s = pl.program_id(0)

    @pl.when(s == 0)
    def _():
        m0T = jax.lax.dot_general(
            w0_ref[0], x0_ref[...], (((0,), (1,)), ((), ())),
            preferred_element_type=jnp.float32)
        m0T_ref[...] = m0T.astype(jnp.bfloat16)

    @pl.when(s < nt0)
    def _():
        col = pl.multiple_of(s * _TM0, _TM0)
        a_bf = a_ref[...].astype(jnp.bfloat16)
        aT = jnp.swapaxes(a_bf, 0, 1)
        aT_ref[:, pl.ds(col, _TM0)] = aT.astype(jnp.int8)
        h = jax.lax.dot_general(m0T_ref[...], aT, (((1,), (0,)), ((), ())),
                                preferred_element_type=jnp.float32)
        sT_ref[:, pl.ds(col, _TM0)] = jnp.maximum(h, 0.0).astype(jnp.bfloat16)

    @pl.when(s < nx1)
    def _():
        row = pl.multiple_of(s * _TX1, _TX1)
        m1 = jnp.dot(x1_ref[...].astype(jnp.bfloat16),
                     w12_ref[...].astype(jnp.bfloat16),
                     preferred_element_type=jnp.float32)
        m1_ref[pl.ds(row, _TX1), :] = m1.astype(jnp.bfloat16)

    rel = s - nt0
    in_p2 = s >= nt0
    is_chunk = jnp.logical_and(in_p2, rel % 2 == 0)

    @pl.when(jnp.logical_and(is_chunk, rel % (2 * nc) == 0))
    def _():
        m0T = jax.lax.dot_general(
            w0_ref[0], sT_ref[...], (((0,), (0,)), ((), ())),
            preferred_element_type=jnp.float32)
        m0T_ref[...] = m0T.astype(jnp.bfloat16)

    @pl.when(jnp.logical_and(is_chunk, rel < 2 * nc))
    def _():
        col = pl.multiple_of(((rel // 2) % nc) * _TC, _TC)
        a_chunk = aT_ref[:, pl.ds(col, _TC)].astype(jnp.bfloat16)
        h = jax.lax.dot_general(m0T_ref[...], a_chunk,
                                (((1,), (0,)), ((), ())),
                                preferred_element_type=jnp.float32)
        sT_ref[:, pl.ds(col, _TC)] = jnp.maximum(h, 0.0).astype(jnp.bfloat16)

    @pl.when(jnp.logical_and(is_chunk, rel >= 2 * nc))
    def _():
        col = pl.multiple_of(((rel // 2) % nc) * _TC, _TC)
        a_chunk = aT_ref[:, pl.ds(col, _TC)].astype(jnp.bfloat16)
        h = jax.lax.dot_general(m0T_ref[...], a_chunk,
                                (((1,), (0,)), ((), ())),
                                preferred_element_type=jnp.float32)
        o1_ref[...] = jnp.swapaxes(jnp.maximum(h, 0.0), 0, 1)

    @pl.when(in_p2)
    def _():
        h = jnp.dot(inc_ref[...].astype(jnp.bfloat16), m1_ref[...],
                    preferred_element_type=jnp.float32)
        o2_ref[...] = jnp.maximum(h, 0.0)


def kernel(x_0, x_1, adjacency_0, incidence_2_t, w0_stack, w12_stack):
    n_nodes, c0 = x_0.shape
    n_edges, c1 = x_1.shape
    n_faces = incidence_2_t.shape[0]
    n_layers = w0_stack.shape[0]
    c2 = w12_stack.shape[2]

    nt0 = n_nodes // _TM0
    nc = n_nodes // _TC
    nx1 = n_edges // _TX1
    ntf = n_faces // _TF
    np2 = ntf
    n_steps = nt0 + np2

    def _w0_idx(s):
        rel = jnp.maximum(s - nt0, 0)
        return jnp.where(s < nt0, 0,
                         jnp.minimum(rel // (2 * nc) + 1, n_layers - 1))

    def _inc_idx(s):
        rel = jnp.maximum(s - nt0, 0)
        return jnp.minimum(rel, ntf - 1)

    def _o1_idx(s):
        rel = jnp.maximum(s - nt0, 0)
        return jnp.clip((rel - 2 * nc) // 2, 0, nc - 1)

    x0_out, x2_out = pl.pallas_call(
        functools.partial(_ccxn_kernel, nt0=nt0, nc=nc, nx1=nx1),
        grid=(n_steps,),
        out_shape=(
            jax.ShapeDtypeStruct((n_nodes, c0), x_0.dtype),
            jax.ShapeDtypeStruct((n_faces, c2), x_1.dtype),
        ),
        in_specs=[
            pl.BlockSpec((n_nodes, c0), lambda s: (0, 0)),
            pl.BlockSpec((1, c0, c0), lambda s: (_w0_idx(s), 0, 0)),
            pl.BlockSpec((_TM0, n_nodes),
                         lambda s: (jnp.minimum(s, nt0 - 1), 0)),
            pl.BlockSpec((_TX1, c1),
                         lambda s: (jnp.minimum(s, nx1 - 1), 0)),
            pl.BlockSpec((c1, c2), lambda s: (0, 0)),
            pl.BlockSpec((_TF, n_edges), lambda s: (_inc_idx(s), 0)),
        ],
        out_specs=(
            pl.BlockSpec((_TC, c0), lambda s: (_o1_idx(s), 0)),
            pl.BlockSpec((_TF, c2), lambda s: (_inc_idx(s), 0)),
        ),
        scratch_shapes=[
            pltpu.VMEM((n_nodes, n_nodes), jnp.int8),
            pltpu.VMEM((c0, n_nodes), jnp.bfloat16),
            pltpu.VMEM((c0, n_nodes), jnp.bfloat16),
            pltpu.VMEM((n_edges, c2), jnp.bfloat16),
        ],
        compiler_params=pltpu.CompilerParams(
            dimension_semantics=("arbitrary",)),
    )(x_0, w0_stack, adjacency_0, x_1, w12_stack[n_layers - 1], incidence_2_t)

    return x0_out, x_1, x2_out
